# Initial kernel scaffold; baseline (speedup 1.0000x reference)
import jax
import jax.numpy as jnp
from jax.experimental import pallas as pl


def kernel(node_attrs, positions, edge_index, params):
    raise NotImplementedError("write your pallas kernel here")


# speedup vs baseline: 3.3938x; 3.3938x over previous
import functools

import jax
import jax.numpy as jnp
from jax import lax
from jax.experimental import pallas as pl
from jax.experimental.pallas import tpu as pltpu
from jax.experimental.pallas import tpu_sc as plsc

f32 = jnp.float32

N_NODES = 100000
N_PAD = 100352
E_EDGES = 1600000
E_PAD = 1638400
BN = 1024
GN = N_PAD // BN
BE = 2048
GE = E_PAD // BE
NC, NS = 2, 16
PER_TILE = E_PAD // (NC * NS)
K_SUB = 16
N_ITERS = 25
BT = K_SUB * 128
KS_S = 8
NI_S = 50
BTS = KS_S * 128
TROWS = N_PAD // NS
ZCH = TROWS // 16


def _silu(v):
    return v * jax.nn.sigmoid(v)


def _prep_body(na, pos, einT, projT, w1aT, w1bT, w1dT, gr_o, gc_o, h_o, x_o):
    h = jnp.dot(na[...], einT[...], preferred_element_type=f32)
    x = jnp.dot(pos[...], projT[...], preferred_element_type=f32)
    pw = jnp.dot(pos[...], w1dT[...], preferred_element_type=f32)
    tr = jnp.dot(h, w1aT[...], preferred_element_type=f32) + pw
    tc = jnp.dot(h, w1bT[...], preferred_element_type=f32) - pw
    gr_o[...] = jnp.concatenate([tr, x], axis=1)
    gc_o[...] = jnp.concatenate([tc, x], axis=1)
    h_o[...] = h
    x_o[...] = x


def _edge_body(gr, gc, w2T, wc1T, vecs, ef0_o, ef1_o, tr_o):
    grv = gr[...]
    gcv = gc[...]
    v = vecs[...]
    xd = grv[:, 32:48] - gcv[:, 32:48]
    radial = jnp.sum(xd * xd, axis=1, keepdims=True)
    s = grv[:, :32] + gcv[:, :32] + radial * v[3:4, :] + v[0:1, :]
    e1 = _silu(s)
    ef = _silu(jnp.dot(e1, w2T[...], preferred_element_type=f32) + v[1:2, :])
    c1 = _silu(jnp.dot(ef, wc1T[...], preferred_element_type=f32) + v[2:3, :])
    cm = jnp.sum(c1 * v[4:5, :], axis=1, keepdims=True)
    tr_o[...] = xd * cm
    ef0_o[...] = ef[:, :16]
    ef1_o[...] = ef[:, 16:32]


def _edge_last_body(gr, gc, w2T, wc1T, vecs, tr_o):
    grv = gr[...]
    gcv = gc[...]
    v = vecs[...]
    xd = grv[:, 32:48] - gcv[:, 32:48]
    radial = jnp.sum(xd * xd, axis=1, keepdims=True)
    s = grv[:, :32] + gcv[:, :32] + radial * v[3:4, :] + v[0:1, :]
    e1 = _silu(s)
    ef = _silu(jnp.dot(e1, w2T[...], preferred_element_type=f32) + v[1:2, :])
    c1 = _silu(jnp.dot(ef, wc1T[...], preferred_element_type=f32) + v[2:3, :])
    cm = jnp.sum(c1 * v[4:5, :], axis=1, keepdims=True)
    tr_o[...] = xd * cm


def _node_body(h, x, pos, P, wn1T, wn2T, nvecs, w1aT, w1bT, w1dT,
               gr_o, gc_o, x1_o, deg_o):
    p = P[...]
    agg_ef = jnp.concatenate([p[0, 0] + p[1, 0], p[0, 1] + p[1, 1]], axis=1)
    agg_tr = p[0, 2] + p[1, 2]
    deg = jnp.maximum(p[0, 3] + p[1, 3], 1.0)
    hv = h[...]
    x1 = x[...] + agg_tr / deg
    nin = jnp.concatenate([hv, agg_ef], axis=1)
    nv = nvecs[...]
    nf = _silu(jnp.dot(nin, wn1T[...], preferred_element_type=f32) + nv[0:1, :])
    h1 = hv + jnp.dot(nf, wn2T[...], preferred_element_type=f32) + nv[1:2, :]
    pw = jnp.dot(pos[...], w1dT[...], preferred_element_type=f32)
    tr = jnp.dot(h1, w1aT[...], preferred_element_type=f32) + pw
    tc = jnp.dot(h1, w1bT[...], preferred_element_type=f32) - pw
    gr_o[...] = jnp.concatenate([tr, x1], axis=1)
    gc_o[...] = jnp.concatenate([tc, x1], axis=1)
    x1_o[...] = x1
    deg_o[...] = deg


def _final_body(x1, P2, deg, outWT, o):
    p = P2[...]
    x2 = x1[...] + (p[0, 0] + p[1, 0]) / deg[...]
    o[...] = jnp.dot(x2, outWT[...], preferred_element_type=f32)


def _rep(shape):
    return pl.BlockSpec(shape, lambda i: tuple(0 for _ in shape))


def _blk(c):
    return pl.BlockSpec((BN, c), lambda i: (i, 0))


def _eblk(c):
    return pl.BlockSpec((BE, c), lambda i: (i, 0))


_prep_call = pl.pallas_call(
    _prep_body, grid=(GN,),
    in_specs=[_blk(8), _blk(8), _rep((8, 32)), _rep((8, 16)),
              _rep((32, 32)), _rep((32, 32)), _rep((8, 32))],
    out_specs=[_blk(48), _blk(48), _blk(32), _blk(16)],
    out_shape=[jax.ShapeDtypeStruct((N_PAD, 48), f32),
               jax.ShapeDtypeStruct((N_PAD, 48), f32),
               jax.ShapeDtypeStruct((N_PAD, 32), f32),
               jax.ShapeDtypeStruct((N_PAD, 16), f32)],
)

_edge_call = pl.pallas_call(
    _edge_body, grid=(GE,),
    in_specs=[_eblk(48), _eblk(48), _rep((32, 32)), _rep((32, 32)),
              _rep((8, 32))],
    out_specs=[_eblk(16), _eblk(16), _eblk(16)],
    out_shape=[jax.ShapeDtypeStruct((E_PAD, 16), f32)] * 3,
)

_edge_last_call = pl.pallas_call(
    _edge_last_body, grid=(GE,),
    in_specs=[_eblk(48), _eblk(48), _rep((32, 32)), _rep((32, 32)),
              _rep((8, 32))],
    out_specs=[_eblk(16)],
    out_shape=[jax.ShapeDtypeStruct((E_PAD, 16), f32)],
)

_node_call = pl.pallas_call(
    _node_body, grid=(GN,),
    in_specs=[_blk(32), _blk(16), _blk(8),
              pl.BlockSpec((2, 4, BN, 16), lambda i: (0, 0, i, 0)),
              _rep((64, 32)), _rep((32, 32)), _rep((8, 32)),
              _rep((32, 32)), _rep((32, 32)), _rep((8, 32))],
    out_specs=[_blk(48), _blk(48), _blk(16), _blk(16)],
    out_shape=[jax.ShapeDtypeStruct((N_PAD, 48), f32),
               jax.ShapeDtypeStruct((N_PAD, 48), f32),
               jax.ShapeDtypeStruct((N_PAD, 16), f32),
               jax.ShapeDtypeStruct((N_PAD, 16), f32)],
)

_final_call = pl.pallas_call(
    _final_body, grid=(GN,),
    in_specs=[_blk(16), pl.BlockSpec((2, 1, BN, 16), lambda i: (0, 0, i, 0)),
              _blk(16), _rep((16, 8))],
    out_specs=[_blk(8)],
    out_shape=[jax.ShapeDtypeStruct((N_PAD, 8), f32)],
)


@functools.cache
def _sc_mesh():
    return plsc.VectorSubcoreMesh(
        core_axis_name="c", subcore_axis_name="s",
        num_cores=NC, num_subcores=NS)


_SC_PARAMS = pltpu.CompilerParams(use_tc_tiling_on_sc=False)


@functools.cache
def _make_gather():
    return pl.kernel(
        _gather_body,
        out_type=[jax.ShapeDtypeStruct((E_PAD, 48), f32),
                  jax.ShapeDtypeStruct((E_PAD, 48), f32)],
        mesh=_sc_mesh(),
        scratch_types=[pltpu.VMEM((K_SUB, 128), jnp.int32),
                       pltpu.VMEM((K_SUB, 128), jnp.int32),
                       pltpu.VMEM((BT, 48), f32),
                       pltpu.SemaphoreType.DMA],
        compiler_params=_SC_PARAMS,
    )


def _gather_body(tab_r, tab_c, idx_r, idx_c, out_r, out_c,
                 idxr_v, idxc_v, rows_v, sem):
    cid = lax.axis_index("c")
    sid = lax.axis_index("s")
    wid = cid * NS + sid
    rbase = wid * (PER_TILE // 128)
    ebase = wid * PER_TILE

    def body(i, carry):
        r0 = rbase + i * K_SUB
        e0 = ebase + i * BT
        pltpu.sync_copy(idx_r.at[pl.ds(r0, K_SUB)], idxr_v)
        pltpu.sync_copy(idx_c.at[pl.ds(r0, K_SUB)], idxc_v)
        descs = [pltpu.async_copy(tab_r.at[idxr_v.at[j]],
                                  rows_v.at[pl.ds(j * 128, 128)], sem)
                 for j in range(K_SUB)]
        for d in descs:
            d.wait()
        pltpu.sync_copy(rows_v, out_r.at[pl.ds(e0, BT)])
        descs = [pltpu.async_copy(tab_c.at[idxc_v.at[j]],
                                  rows_v.at[pl.ds(j * 128, 128)], sem)
                 for j in range(K_SUB)]
        for d in descs:
            d.wait()
        pltpu.sync_copy(rows_v, out_c.at[pl.ds(e0, BT)])
        return carry

    lax.fori_loop(0, N_ITERS, body, 0)


@functools.cache
def _make_scatter(nvals, with_deg):
    npass = nvals + (1 if with_deg else 0)
    scratch = [pltpu.VMEM((KS_S, 128), jnp.int32),
               pltpu.VMEM((BTS, 16), f32),
               pltpu.VMEM((ZCH, 16), f32),
               pltpu.VMEM((128, 16), f32),
               pltpu.VMEM_SHARED((N_PAD, 16), f32),
               pltpu.SemaphoreType.DMA]

    @functools.partial(
        pl.kernel,
        out_type=[jax.ShapeDtypeStruct((NC, npass, N_PAD, 16), f32)],
        mesh=_sc_mesh(),
        scratch_types=scratch,
        compiler_params=_SC_PARAMS,
    )
    def _scatter_kernel(*args):
        idx_s = args[0]
        vals_hbm = args[1:1 + nvals]
        out = args[1 + nvals]
        idx_v, vals_v, zbuf, ones_v, acc, sem = args[2 + nvals:]
        cid = lax.axis_index("c")
        sid = lax.axis_index("s")
        wid = cid * NS + sid
        rbase = wid * (PER_TILE // 128)
        ebase = wid * PER_TILE

        def zfill(i, carry):
            zbuf[i, :] = jnp.zeros((16,), f32)
            return carry
        lax.fori_loop(0, ZCH, zfill, 0)
        if with_deg:
            def ofill(i, carry):
                ones_v[i, :] = jnp.full((16,), 1.0, f32)
                return carry
            lax.fori_loop(0, 128, ofill, 0)

        for p in range(npass):
            is_deg = with_deg and p == nvals

            def zero(z, carry):
                pltpu.sync_copy(zbuf, acc.at[pl.ds(sid * TROWS + z * ZCH, ZCH)])
                return carry
            lax.fori_loop(0, 16, zero, 0)
            plsc.subcore_barrier()

            def body(i, carry):
                r0 = rbase + i * KS_S
                e0 = ebase + i * BTS
                pltpu.sync_copy(idx_s.at[pl.ds(r0, KS_S)], idx_v)
                if not is_deg:
                    pltpu.sync_copy(vals_hbm[p].at[pl.ds(e0, BTS)], vals_v)
                for j in range(KS_S):
                    src = ones_v if is_deg else vals_v.at[pl.ds(j * 128, 128)]
                    pltpu.sync_copy(src, acc.at[idx_v.at[j]], add=True)
                return carry
            lax.fori_loop(0, NI_S, body, 0)
            plsc.subcore_barrier()
            pltpu.sync_copy(acc.at[pl.ds(sid * TROWS, TROWS)],
                            out.at[cid, p, pl.ds(sid * TROWS, TROWS)])
            plsc.subcore_barrier()

    return _scatter_kernel


def kernel(node_attrs, positions, edge_index, params):
    prm = params
    l0, l1 = prm["layers"][0], prm["layers"][1]

    na = jnp.zeros((N_PAD, 8), f32)
    na = na.at[:N_NODES, :3].set(node_attrs.astype(f32))
    na = na.at[:N_NODES, 3].set(1.0)
    pos = jnp.zeros((N_PAD, 8), f32)
    pos = pos.at[:N_NODES, :3].set(positions.astype(f32))
    pos = pos.at[:N_NODES, 3].set(1.0)

    row = edge_index[0].astype(jnp.int32)
    col = edge_index[1].astype(jnp.int32)
    pad_e = E_PAD - E_EDGES
    row_g = jnp.concatenate([row, jnp.zeros((pad_e,), jnp.int32)])
    col_g = jnp.concatenate([col, jnp.zeros((pad_e,), jnp.int32)])
    row_s = jnp.concatenate(
        [row, jnp.full((pad_e,), N_NODES, jnp.int32)])
    row_g = row_g.reshape(E_PAD // 128, 128)
    col_g = col_g.reshape(E_PAD // 128, 128)
    row_s = row_s.reshape(E_PAD // 128, 128)

    def vpad(v, rows=8):
        out = jnp.zeros((rows, v.shape[1]), f32)
        return out.at[:v.shape[0]].set(v)

    einT = vpad(jnp.concatenate([prm["emb_in_w"].T,
                                 prm["emb_in_b"][None, :]], axis=0))
    projT = vpad(prm["proj_w"].T)
    outWT = jnp.zeros((16, 8), f32).at[:, :3].set(prm["out_w"].T)

    def layer_mats(lp):
        w1 = lp["edge_w1"]
        w1aT = w1[:, :32].T
        w1bT = w1[:, 32:64].T
        w1dT = vpad(w1[:, 65:68].T)
        vecs = jnp.zeros((8, 32), f32)
        vecs = vecs.at[0].set(lp["edge_b1"])
        vecs = vecs.at[1].set(lp["edge_b2"])
        vecs = vecs.at[2].set(lp["coord_b1"])
        vecs = vecs.at[3].set(w1[:, 64])
        vecs = vecs.at[4].set(lp["coord_w2"][0])
        return w1aT, w1bT, w1dT, vecs, lp["edge_w2"].T, lp["coord_w1"].T

    w1aT0, w1bT0, w1dT0, vecs0, w2T0, wc1T0 = layer_mats(l0)
    w1aT1, w1bT1, w1dT1, vecs1, w2T1, wc1T1 = layer_mats(l1)
    wn1T0 = l0["node_w1"].T
    wn2T0 = l0["node_w2"].T
    nvecs0 = jnp.zeros((8, 32), f32)
    nvecs0 = nvecs0.at[0].set(l0["node_b1"]).at[1].set(l0["node_b2"])

    gr_tab0, gc_tab0, h0, x0 = _prep_call(
        na, pos, einT, projT, w1aT0, w1bT0, w1dT0)
    gather = _make_gather()
    gr_e0, gc_e0 = gather(gr_tab0, gc_tab0, row_g, col_g)
    ef0a, ef0b, tr0 = _edge_call(gr_e0, gc_e0, w2T0, wc1T0, vecs0)
    (p0,) = _make_scatter(3, True)(row_s, ef0a, ef0b, tr0)
    gr_tab1, gc_tab1, x1, deg = _node_call(
        h0, x0, pos, p0, wn1T0, wn2T0, nvecs0, w1aT1, w1bT1, w1dT1)
    gr_e1, gc_e1 = gather(gr_tab1, gc_tab1, row_g, col_g)
    (tr1,) = _edge_last_call(gr_e1, gc_e1, w2T1, wc1T1, vecs1)
    (p1,) = _make_scatter(1, False)(row_s, tr1)
    (out,) = _final_call(x1, p1, deg, outWT)
    return out[:N_NODES, :3]


# speedup vs baseline: 4.4506x; 1.3114x over previous
import functools

import jax
import jax.numpy as jnp
from jax import lax
from jax.experimental import pallas as pl
from jax.experimental.pallas import tpu as pltpu
from jax.experimental.pallas import tpu_sc as plsc

f32 = jnp.float32

N_NODES = 100000
N_PAD = 100352
E_EDGES = 1600000
E_PAD = 1638400
BN = 1024
GN = N_PAD // BN
BE = 2048
GE = E_PAD // BE
NC, NS = 2, 16
PER_TILE = E_PAD // (NC * NS)
K_G = 8
NI_G = 50
BTG = K_G * 128
NB_T = PER_TILE // BE
KS_S = 8
NI_S = 50
BTS = KS_S * 128
TROWS = N_PAD // NS
ZCH = TROWS // 16


def _silu(v):
    return v * jax.nn.sigmoid(v)


def _prep_body(na, pos, einT, projT, w1aT, w1bT, w1dT, gr_o, gc_o, h_o, x_o):
    h = jnp.dot(na[...], einT[...], preferred_element_type=f32)
    x = jnp.dot(pos[...], projT[...], preferred_element_type=f32)
    pw = jnp.dot(pos[...], w1dT[...], preferred_element_type=f32)
    tr = jnp.dot(h, w1aT[...], preferred_element_type=f32) + pw
    tc = jnp.dot(h, w1bT[...], preferred_element_type=f32) - pw
    gr_o[...] = jnp.concatenate([tr, x], axis=1)
    gc_o[...] = jnp.concatenate([tc, x], axis=1)
    h_o[...] = h
    x_o[...] = x


def _edge_math(g, v, w2T, wc1T, want_ef):
    xd = g[:, 32:48] - g[:, 80:96]
    radial = jnp.sum(xd * xd, axis=1, keepdims=True)
    s = g[:, 0:32] + g[:, 48:80] + radial * v[3:4, :] + v[0:1, :]
    e1 = _silu(s)
    ef = _silu(jnp.dot(e1, w2T, preferred_element_type=f32) + v[1:2, :])
    c1 = _silu(jnp.dot(ef, wc1T, preferred_element_type=f32) + v[2:3, :])
    cm = jnp.sum(c1 * v[4:5, :], axis=1, keepdims=True)
    tr = xd * cm
    if want_ef:
        return jnp.concatenate(
            [ef, tr, jnp.zeros((g.shape[0], 16), f32)], axis=1)
    return tr


def _edge_body(ga, gb, w2T, wc1T, vecs, out_o):
    v = vecs[...]
    ra = _edge_math(ga[...], v, w2T[...], wc1T[...], True)
    rb = _edge_math(gb[...], v, w2T[...], wc1T[...], True)
    out_o[...] = jnp.concatenate([ra, rb], axis=1)


def _edge_last_body(ga, gb, w2T, wc1T, vecs, tr_o):
    v = vecs[...]
    ta = _edge_math(ga[...], v, w2T[...], wc1T[...], False)
    tb = _edge_math(gb[...], v, w2T[...], wc1T[...], False)
    tr_o[...] = jnp.concatenate([ta, tb], axis=0)


def _node_body(h, x, pos, P, wn1T, wn2T, nvecs, w1aT, w1bT, w1dT,
               gr_o, gc_o, x1_o, deg_o):
    p = P[...]
    agg_ef = jnp.concatenate([p[0, 0] + p[1, 0], p[0, 1] + p[1, 1]], axis=1)
    agg_tr = p[0, 2] + p[1, 2]
    deg = jnp.maximum(p[0, 3] + p[1, 3], 1.0)
    hv = h[...]
    x1 = x[...] + agg_tr / deg
    nin = jnp.concatenate([hv, agg_ef], axis=1)
    nv = nvecs[...]
    nf = _silu(jnp.dot(nin, wn1T[...], preferred_element_type=f32) + nv[0:1, :])
    h1 = hv + jnp.dot(nf, wn2T[...], preferred_element_type=f32) + nv[1:2, :]
    pw = jnp.dot(pos[...], w1dT[...], preferred_element_type=f32)
    tr = jnp.dot(h1, w1aT[...], preferred_element_type=f32) + pw
    tc = jnp.dot(h1, w1bT[...], preferred_element_type=f32) - pw
    gr_o[...] = jnp.concatenate([tr, x1], axis=1)
    gc_o[...] = jnp.concatenate([tc, x1], axis=1)
    x1_o[...] = x1
    deg_o[...] = deg


def _final_body(x1, P2, deg, outWT, o):
    p = P2[...]
    x2 = x1[...] + (p[0, 0] + p[1, 0]) / deg[...]
    o[...] = jnp.dot(x2, outWT[...], preferred_element_type=f32)


def _rep(shape):
    return pl.BlockSpec(shape, lambda i: tuple(0 for _ in shape))


def _blk(c):
    return pl.BlockSpec((BN, c), lambda i: (i, 0))


def _eblk(c):
    return pl.BlockSpec((BE, c), lambda i: (i, 0))


_prep_call = pl.pallas_call(
    _prep_body, grid=(GN,),
    in_specs=[_blk(8), _blk(8), _rep((8, 32)), _rep((8, 16)),
              _rep((32, 32)), _rep((32, 32)), _rep((8, 32))],
    out_specs=[_blk(48), _blk(48), _blk(32), _blk(16)],
    out_shape=[jax.ShapeDtypeStruct((N_PAD, 48), f32),
               jax.ShapeDtypeStruct((N_PAD, 48), f32),
               jax.ShapeDtypeStruct((N_PAD, 32), f32),
               jax.ShapeDtypeStruct((N_PAD, 16), f32)],
)

_HB = BE // 2

_edge_call = pl.pallas_call(
    _edge_body, grid=(GE,),
    in_specs=[pl.BlockSpec((_HB, 128), lambda i: (2 * i, 0)),
              pl.BlockSpec((_HB, 128), lambda i: (2 * i + 1, 0)),
              _rep((32, 32)), _rep((32, 32)), _rep((8, 32))],
    out_specs=[pl.BlockSpec((_HB, 128), lambda i: (i, 0))],
    out_shape=[jax.ShapeDtypeStruct((E_PAD // 2, 128), f32)],
)

_edge_last_call = pl.pallas_call(
    _edge_last_body, grid=(GE,),
    in_specs=[pl.BlockSpec((_HB, 128), lambda i: (2 * i, 0)),
              pl.BlockSpec((_HB, 128), lambda i: (2 * i + 1, 0)),
              _rep((32, 32)), _rep((32, 32)), _rep((8, 32))],
    out_specs=[_eblk(16)],
    out_shape=[jax.ShapeDtypeStruct((E_PAD, 16), f32)],
)

_node_call = pl.pallas_call(
    _node_body, grid=(GN,),
    in_specs=[_blk(32), _blk(16), _blk(8),
              pl.BlockSpec((2, 4, BN, 16), lambda i: (0, 0, i, 0)),
              _rep((64, 32)), _rep((32, 32)), _rep((8, 32)),
              _rep((32, 32)), _rep((32, 32)), _rep((8, 32))],
    out_specs=[_blk(48), _blk(48), _blk(16), _blk(16)],
    out_shape=[jax.ShapeDtypeStruct((N_PAD, 48), f32),
               jax.ShapeDtypeStruct((N_PAD, 48), f32),
               jax.ShapeDtypeStruct((N_PAD, 16), f32),
               jax.ShapeDtypeStruct((N_PAD, 16), f32)],
)

_final_call = pl.pallas_call(
    _final_body, grid=(GN,),
    in_specs=[_blk(16), pl.BlockSpec((2, 1, BN, 16), lambda i: (0, 0, i, 0)),
              _blk(16), _rep((16, 8))],
    out_specs=[_blk(8)],
    out_shape=[jax.ShapeDtypeStruct((N_PAD, 8), f32)],
)


@functools.cache
def _sc_mesh():
    return plsc.VectorSubcoreMesh(
        core_axis_name="c", subcore_axis_name="s",
        num_cores=NC, num_subcores=NS)


_SC_PARAMS = pltpu.CompilerParams(use_tc_tiling_on_sc=False)


@functools.cache
def _make_gather():
    return pl.kernel(
        _gather_body,
        out_type=[jax.ShapeDtypeStruct((E_PAD, 128), f32)],
        mesh=_sc_mesh(),
        scratch_types=[pltpu.VMEM((K_G, 128), jnp.int32),
                       pltpu.VMEM((K_G, 128), jnp.int32),
                       pltpu.VMEM((BTG, 48), f32),
                       pltpu.VMEM((BTG, 48), f32),
                       pltpu.SemaphoreType.DMA],
        compiler_params=_SC_PARAMS,
    )


def _gather_body(tab_r, tab_c, idx_r, idx_c, out,
                 idxr_v, idxc_v, rows_r, rows_c, sem):
    cid = lax.axis_index("c")
    sid = lax.axis_index("s")
    wid = cid * NS + sid
    rbase = wid * (PER_TILE // 128)
    ebase = wid * PER_TILE

    def body(i, carry):
        r0 = rbase + i * K_G
        e0 = ebase + i * BTG
        pltpu.sync_copy(idx_r.at[pl.ds(r0, K_G)], idxr_v)
        pltpu.sync_copy(idx_c.at[pl.ds(r0, K_G)], idxc_v)
        da = [pltpu.async_copy(tab_r.at[idxr_v.at[j]],
                               rows_r.at[pl.ds(j * 128, 128)], sem)
              for j in range(K_G)]
        db = [pltpu.async_copy(tab_c.at[idxc_v.at[j]],
                               rows_c.at[pl.ds(j * 128, 128)], sem)
              for j in range(K_G)]
        for d in da:
            d.wait()
        for d in db:
            d.wait()
        pltpu.sync_copy(rows_r, out.at[pl.ds(e0, BTG), pl.ds(0, 48)])
        pltpu.sync_copy(rows_c, out.at[pl.ds(e0, BTG), pl.ds(48, 48)])
        return carry

    lax.fori_loop(0, NI_G, body, 0)


@functools.cache
def _make_scatter_paired():
    scratch = [pltpu.VMEM((16, 128), jnp.int32),
               pltpu.VMEM((BE // 2, 16), f32),
               pltpu.VMEM((ZCH, 16), f32),
               pltpu.VMEM((128, 16), f32),
               pltpu.VMEM_SHARED((N_PAD, 16), f32),
               pltpu.SemaphoreType.DMA]

    @functools.partial(
        pl.kernel,
        out_type=[jax.ShapeDtypeStruct((NC, 4, N_PAD, 16), f32)],
        mesh=_sc_mesh(),
        scratch_types=scratch,
        compiler_params=_SC_PARAMS,
    )
    def _scatter_kernel(idx_s, vals, out, idx_v, vals_v, zbuf, ones_v, acc,
                        sem):
        cid = lax.axis_index("c")
        sid = lax.axis_index("s")
        wid = cid * NS + sid
        tbase = wid * NB_T

        def zfill(i, carry):
            zbuf[i, :] = jnp.zeros((16,), f32)
            return carry
        lax.fori_loop(0, ZCH, zfill, 0)

        def ofill(i, carry):
            ones_v[i, :] = jnp.full((16,), 1.0, f32)
            return carry
        lax.fori_loop(0, 128, ofill, 0)

        for p in range(4):
            is_deg = p == 3

            def zero(z, carry):
                pltpu.sync_copy(zbuf, acc.at[pl.ds(sid * TROWS + z * ZCH, ZCH)])
                return carry
            lax.fori_loop(0, 16, zero, 0)
            plsc.subcore_barrier()

            def body(i, carry):
                gb = tbase + i
                pltpu.sync_copy(idx_s.at[pl.ds(16 * gb, 16)], idx_v)
                if not is_deg:
                    pltpu.sync_copy(
                        vals.at[pl.ds(BE // 2 * gb, BE // 2),
                                pl.ds(16 * p, 16)], vals_v)
                for k in range(8):
                    src = ones_v if is_deg else vals_v.at[pl.ds(k * 128, 128)]
                    pltpu.sync_copy(src, acc.at[idx_v.at[k]], add=True)
                if not is_deg:
                    pltpu.sync_copy(
                        vals.at[pl.ds(BE // 2 * gb, BE // 2),
                                pl.ds(64 + 16 * p, 16)], vals_v)
                for k in range(8):
                    src = ones_v if is_deg else vals_v.at[pl.ds(k * 128, 128)]
                    pltpu.sync_copy(src, acc.at[idx_v.at[8 + k]], add=True)
                return carry
            lax.fori_loop(0, NB_T, body, 0)
            plsc.subcore_barrier()
            pltpu.sync_copy(acc.at[pl.ds(sid * TROWS, TROWS)],
                            out.at[cid, p, pl.ds(sid * TROWS, TROWS)])
            plsc.subcore_barrier()

    return _scatter_kernel


@functools.cache
def _make_scatter(nvals, with_deg):
    npass = nvals + (1 if with_deg else 0)
    scratch = [pltpu.VMEM((KS_S, 128), jnp.int32),
               pltpu.VMEM((BTS, 16), f32),
               pltpu.VMEM((ZCH, 16), f32),
               pltpu.VMEM((128, 16), f32),
               pltpu.VMEM_SHARED((N_PAD, 16), f32),
               pltpu.SemaphoreType.DMA]

    @functools.partial(
        pl.kernel,
        out_type=[jax.ShapeDtypeStruct((NC, npass, N_PAD, 16), f32)],
        mesh=_sc_mesh(),
        scratch_types=scratch,
        compiler_params=_SC_PARAMS,
    )
    def _scatter_kernel(*args):
        idx_s = args[0]
        vals_hbm = args[1:1 + nvals]
        out = args[1 + nvals]
        idx_v, vals_v, zbuf, ones_v, acc, sem = args[2 + nvals:]
        cid = lax.axis_index("c")
        sid = lax.axis_index("s")
        wid = cid * NS + sid
        rbase = wid * (PER_TILE // 128)
        ebase = wid * PER_TILE

        def zfill(i, carry):
            zbuf[i, :] = jnp.zeros((16,), f32)
            return carry
        lax.fori_loop(0, ZCH, zfill, 0)
        if with_deg:
            def ofill(i, carry):
                ones_v[i, :] = jnp.full((16,), 1.0, f32)
                return carry
            lax.fori_loop(0, 128, ofill, 0)

        for p in range(npass):
            is_deg = with_deg and p == nvals

            def zero(z, carry):
                pltpu.sync_copy(zbuf, acc.at[pl.ds(sid * TROWS + z * ZCH, ZCH)])
                return carry
            lax.fori_loop(0, 16, zero, 0)
            plsc.subcore_barrier()

            def body(i, carry):
                r0 = rbase + i * KS_S
                e0 = ebase + i * BTS
                pltpu.sync_copy(idx_s.at[pl.ds(r0, KS_S)], idx_v)
                if not is_deg:
                    pltpu.sync_copy(vals_hbm[p].at[pl.ds(e0, BTS)], vals_v)
                for j in range(KS_S):
                    src = ones_v if is_deg else vals_v.at[pl.ds(j * 128, 128)]
                    pltpu.sync_copy(src, acc.at[idx_v.at[j]], add=True)
                return carry
            lax.fori_loop(0, NI_S, body, 0)
            plsc.subcore_barrier()
            pltpu.sync_copy(acc.at[pl.ds(sid * TROWS, TROWS)],
                            out.at[cid, p, pl.ds(sid * TROWS, TROWS)])
            plsc.subcore_barrier()

    return _scatter_kernel


def kernel(node_attrs, positions, edge_index, params):
    prm = params
    l0, l1 = prm["layers"][0], prm["layers"][1]

    na = jnp.zeros((N_PAD, 8), f32)
    na = na.at[:N_NODES, :3].set(node_attrs.astype(f32))
    na = na.at[:N_NODES, 3].set(1.0)
    pos = jnp.zeros((N_PAD, 8), f32)
    pos = pos.at[:N_NODES, :3].set(positions.astype(f32))
    pos = pos.at[:N_NODES, 3].set(1.0)

    row = edge_index[0].astype(jnp.int32)
    col = edge_index[1].astype(jnp.int32)
    pad_e = E_PAD - E_EDGES
    row_g = jnp.concatenate([row, jnp.zeros((pad_e,), jnp.int32)])
    col_g = jnp.concatenate([col, jnp.zeros((pad_e,), jnp.int32)])
    row_s = jnp.concatenate(
        [row, jnp.full((pad_e,), N_NODES, jnp.int32)])
    row_g = row_g.reshape(E_PAD // 128, 128)
    col_g = col_g.reshape(E_PAD // 128, 128)
    row_s = row_s.reshape(E_PAD // 128, 128)

    def vpad(v, rows=8):
        out = jnp.zeros((rows, v.shape[1]), f32)
        return out.at[:v.shape[0]].set(v)

    einT = vpad(jnp.concatenate([prm["emb_in_w"].T,
                                 prm["emb_in_b"][None, :]], axis=0))
    projT = vpad(prm["proj_w"].T)
    outWT = jnp.zeros((16, 8), f32).at[:, :3].set(prm["out_w"].T)

    def layer_mats(lp):
        w1 = lp["edge_w1"]
        w1aT = w1[:, :32].T
        w1bT = w1[:, 32:64].T
        w1dT = vpad(w1[:, 65:68].T)
        vecs = jnp.zeros((8, 32), f32)
        vecs = vecs.at[0].set(lp["edge_b1"])
        vecs = vecs.at[1].set(lp["edge_b2"])
        vecs = vecs.at[2].set(lp["coord_b1"])
        vecs = vecs.at[3].set(w1[:, 64])
        vecs = vecs.at[4].set(lp["coord_w2"][0])
        return w1aT, w1bT, w1dT, vecs, lp["edge_w2"].T, lp["coord_w1"].T

    w1aT0, w1bT0, w1dT0, vecs0, w2T0, wc1T0 = layer_mats(l0)
    w1aT1, w1bT1, w1dT1, vecs1, w2T1, wc1T1 = layer_mats(l1)
    wn1T0 = l0["node_w1"].T
    wn2T0 = l0["node_w2"].T
    nvecs0 = jnp.zeros((8, 32), f32)
    nvecs0 = nvecs0.at[0].set(l0["node_b1"]).at[1].set(l0["node_b2"])

    gr_tab0, gc_tab0, h0, x0 = _prep_call(
        na, pos, einT, projT, w1aT0, w1bT0, w1dT0)
    gather = _make_gather()
    (ge0,) = gather(gr_tab0, gc_tab0, row_g, col_g)
    (rec0,) = _edge_call(ge0, ge0, w2T0, wc1T0, vecs0)
    (p0,) = _make_scatter_paired()(row_s, rec0)
    gr_tab1, gc_tab1, x1, deg = _node_call(
        h0, x0, pos, p0, wn1T0, wn2T0, nvecs0, w1aT1, w1bT1, w1dT1)
    (ge1,) = gather(gr_tab1, gc_tab1, row_g, col_g)
    (tr1,) = _edge_last_call(ge1, ge1, w2T1, wc1T1, vecs1)
    (p1,) = _make_scatter(1, False)(row_s, tr1)
    (out,) = _final_call(x1, p1, deg, outWT)
    return out[:N_NODES, :3]


# speedup vs baseline: 5.3057x; 1.1921x over previous
import functools

import jax
import jax.numpy as jnp
from jax import lax
from jax.experimental import pallas as pl
from jax.experimental.pallas import tpu as pltpu
from jax.experimental.pallas import tpu_sc as plsc

f32 = jnp.float32

N_NODES = 100000
N_PAD = 100352
E_EDGES = 1600000
E_PAD = 1638400
BN = 1024
GN = N_PAD // BN
BE = 2048
GE = E_PAD // BE
NC, NS = 2, 16
PER_TILE = E_PAD // (NC * NS)
K_G = 8
NI_G = 50
BTG = K_G * 128
NB_T = PER_TILE // BE
KS_S = 8
NI_S = 50
BTS = KS_S * 128
TROWS = N_PAD // NS
ZCH = TROWS // 16


def _silu(v):
    return v * jax.nn.sigmoid(v)


def _prep_body(na, pos, einT, projT, w1aT, w1bT, w1dT, gr_o, gc_o, h_o, x_o):
    h = jnp.dot(na[...], einT[...], preferred_element_type=f32)
    x = jnp.dot(pos[...], projT[...], preferred_element_type=f32)
    pw = jnp.dot(pos[...], w1dT[...], preferred_element_type=f32)
    tr = jnp.dot(h, w1aT[...], preferred_element_type=f32) + pw
    tc = jnp.dot(h, w1bT[...], preferred_element_type=f32) - pw
    gr_o[...] = jnp.concatenate([tr, x], axis=1)
    gc_o[...] = jnp.concatenate([tc, x], axis=1)
    h_o[...] = h
    x_o[...] = x


def _edge_math(g, v, w2T, wc1T, want_ef):
    xd = g[:, 32:48] - g[:, 80:96]
    radial = jnp.sum(xd * xd, axis=1, keepdims=True)
    s = g[:, 0:32] + g[:, 48:80] + radial * v[3:4, :] + v[0:1, :]
    e1 = _silu(s).astype(jnp.bfloat16)
    ef = _silu(jnp.dot(e1, w2T, preferred_element_type=f32) + v[1:2, :])
    c1 = _silu(jnp.dot(ef.astype(jnp.bfloat16), wc1T,
                       preferred_element_type=f32) + v[2:3, :])
    cm = jnp.sum(c1 * v[4:5, :], axis=1, keepdims=True)
    tr = xd * cm
    if want_ef:
        return jnp.concatenate(
            [ef, tr, jnp.zeros((g.shape[0], 16), f32)], axis=1)
    return tr


def _edge_body(ga, gb, w2T, wc1T, vecs, out_o):
    v = vecs[...]
    w2b = w2T[...].astype(jnp.bfloat16)
    wc1b = wc1T[...].astype(jnp.bfloat16)
    ra = _edge_math(ga[...], v, w2b, wc1b, True)
    rb = _edge_math(gb[...], v, w2b, wc1b, True)
    out_o[...] = jnp.concatenate([ra, rb], axis=1)


def _edge_last_body(ga, gb, w2T, wc1T, vecs, tr_o):
    v = vecs[...]
    w2b = w2T[...].astype(jnp.bfloat16)
    wc1b = wc1T[...].astype(jnp.bfloat16)
    ta = _edge_math(ga[...], v, w2b, wc1b, False)
    tb = _edge_math(gb[...], v, w2b, wc1b, False)
    tr_o[...] = jnp.concatenate([ta, tb], axis=0)


def _node_body(h, x, pos, P, wn1T, wn2T, nvecs, w1aT, w1bT, w1dT,
               gr_o, gc_o, x1_o, deg_o):
    p = P[...]
    agg_ef = jnp.concatenate([p[0, 0] + p[1, 0], p[0, 1] + p[1, 1]], axis=1)
    agg_tr = p[0, 2] + p[1, 2]
    deg = jnp.maximum(p[0, 3] + p[1, 3], 1.0)
    hv = h[...]
    x1 = x[...] + agg_tr / deg
    nin = jnp.concatenate([hv, agg_ef], axis=1)
    nv = nvecs[...]
    nf = _silu(jnp.dot(nin, wn1T[...], preferred_element_type=f32) + nv[0:1, :])
    h1 = hv + jnp.dot(nf, wn2T[...], preferred_element_type=f32) + nv[1:2, :]
    pw = jnp.dot(pos[...], w1dT[...], preferred_element_type=f32)
    tr = jnp.dot(h1, w1aT[...], preferred_element_type=f32) + pw
    tc = jnp.dot(h1, w1bT[...], preferred_element_type=f32) - pw
    gr_o[...] = jnp.concatenate([tr, x1], axis=1)
    gc_o[...] = jnp.concatenate([tc, x1], axis=1)
    x1_o[...] = x1
    deg_o[...] = deg


def _final_body(x1, P2, deg, outWT, o):
    p = P2[...]
    x2 = x1[...] + (p[0, 0] + p[1, 0]) / deg[...]
    o[...] = jnp.dot(x2, outWT[...], preferred_element_type=f32)


def _rep(shape):
    return pl.BlockSpec(shape, lambda i: tuple(0 for _ in shape))


def _blk(c):
    return pl.BlockSpec((BN, c), lambda i: (i, 0))


def _eblk(c):
    return pl.BlockSpec((BE, c), lambda i: (i, 0))


_prep_call = pl.pallas_call(
    _prep_body, grid=(GN,),
    in_specs=[_blk(8), _blk(8), _rep((8, 32)), _rep((8, 16)),
              _rep((32, 32)), _rep((32, 32)), _rep((8, 32))],
    out_specs=[_blk(48), _blk(48), _blk(32), _blk(16)],
    out_shape=[jax.ShapeDtypeStruct((N_PAD, 48), f32),
               jax.ShapeDtypeStruct((N_PAD, 48), f32),
               jax.ShapeDtypeStruct((N_PAD, 32), f32),
               jax.ShapeDtypeStruct((N_PAD, 16), f32)],
)

_HB = BE // 2

_edge_call = pl.pallas_call(
    _edge_body, grid=(GE,),
    in_specs=[pl.BlockSpec((_HB, 128), lambda i: (2 * i, 0)),
              pl.BlockSpec((_HB, 128), lambda i: (2 * i + 1, 0)),
              _rep((32, 32)), _rep((32, 32)), _rep((8, 32))],
    out_specs=[pl.BlockSpec((_HB, 128), lambda i: (i, 0))],
    out_shape=[jax.ShapeDtypeStruct((E_PAD // 2, 128), f32)],
)

_edge_last_call = pl.pallas_call(
    _edge_last_body, grid=(GE,),
    in_specs=[pl.BlockSpec((_HB, 128), lambda i: (2 * i, 0)),
              pl.BlockSpec((_HB, 128), lambda i: (2 * i + 1, 0)),
              _rep((32, 32)), _rep((32, 32)), _rep((8, 32))],
    out_specs=[_eblk(16)],
    out_shape=[jax.ShapeDtypeStruct((E_PAD, 16), f32)],
)

_node_call = pl.pallas_call(
    _node_body, grid=(GN,),
    in_specs=[_blk(32), _blk(16), _blk(8),
              pl.BlockSpec((2, 4, BN, 16), lambda i: (0, 0, i, 0)),
              _rep((64, 32)), _rep((32, 32)), _rep((8, 32)),
              _rep((32, 32)), _rep((32, 32)), _rep((8, 32))],
    out_specs=[_blk(48), _blk(48), _blk(16), _blk(16)],
    out_shape=[jax.ShapeDtypeStruct((N_PAD, 48), f32),
               jax.ShapeDtypeStruct((N_PAD, 48), f32),
               jax.ShapeDtypeStruct((N_PAD, 16), f32),
               jax.ShapeDtypeStruct((N_PAD, 16), f32)],
)

_final_call = pl.pallas_call(
    _final_body, grid=(GN,),
    in_specs=[_blk(16), pl.BlockSpec((2, 1, BN, 16), lambda i: (0, 0, i, 0)),
              _blk(16), _rep((16, 8))],
    out_specs=[_blk(8)],
    out_shape=[jax.ShapeDtypeStruct((N_PAD, 8), f32)],
)


@functools.cache
def _sc_mesh():
    return plsc.VectorSubcoreMesh(
        core_axis_name="c", subcore_axis_name="s",
        num_cores=NC, num_subcores=NS)


_SC_PARAMS = pltpu.CompilerParams(use_tc_tiling_on_sc=False)


@functools.cache
def _make_gather():
    return pl.kernel(
        _gather_body,
        out_type=[jax.ShapeDtypeStruct((E_PAD, 128), f32)],
        mesh=_sc_mesh(),
        scratch_types=[pltpu.VMEM((BTG, 48), f32),
                       pltpu.VMEM((BTG, 48), f32),
                       pltpu.VMEM((K_G, 128), jnp.int32),
                       pltpu.VMEM((K_G, 128), jnp.int32),
                       pltpu.SemaphoreType.DMA,
                       pltpu.SemaphoreType.DMA,
                       pltpu.SemaphoreType.DMA,
                       pltpu.SemaphoreType.DMA],
        compiler_params=_SC_PARAMS,
    )


_RMAX = E_PAD // 128 - K_G


def _gather_body(tab_r, tab_c, idx_r, idx_c, out,
                 rows0, rows1, idx0, idx1, sem_g, sem_i, sem_w0, sem_w1):
    cid = lax.axis_index("c")
    sid = lax.axis_index("s")
    wid = cid * NS + sid
    rbase = wid * (PER_TILE // 128)
    ebase = wid * PER_TILE
    rows = (rows0, rows1)
    idxb = (idx0, idx1)
    sem_w = (sem_w0, sem_w1)

    def run_pass(tab, idxarr, colbase):
        def fire(i, b):
            for j in range(K_G):
                pltpu.async_copy(tab.at[idxb[b].at[j]],
                                 rows[b].at[pl.ds(j * 128, 128)], sem_g)

        def idx_load(i, b):
            rc = jnp.minimum(rbase + i * K_G, _RMAX)
            pltpu.async_copy(idxarr.at[pl.ds(rc, K_G)], idxb[b], sem_i)

        pltpu.sync_copy(idxarr.at[pl.ds(rbase, K_G)], idx0)
        fire(0, 0)
        idx_load(1, 1)

        def body(i, carry):
            b = lax.rem(i, 2)

            @pl.when(i + 1 < NI_G)
            def _():
                pltpu.make_async_copy(
                    idxarr.at[pl.ds(0, K_G)], idx0, sem_i).wait()

                @pl.when((i >= 1) & (b == 1))
                def _():
                    pltpu.make_async_copy(
                        rows0, out.at[pl.ds(0, BTG), pl.ds(colbase, 48)],
                        sem_w0).wait()

                @pl.when((i >= 1) & (b == 0))
                def _():
                    pltpu.make_async_copy(
                        rows1, out.at[pl.ds(0, BTG), pl.ds(colbase, 48)],
                        sem_w1).wait()

                @pl.when(b == 0)
                def _():
                    fire(i + 1, 1)

                @pl.when(b == 1)
                def _():
                    fire(i + 1, 0)

            pltpu.make_async_copy(
                tab.at[pl.ds(0, BTG)], rows0, sem_g).wait()
            e0 = ebase + i * BTG

            @pl.when(b == 0)
            def _():
                pltpu.async_copy(
                    rows0, out.at[pl.ds(e0, BTG), pl.ds(colbase, 48)], sem_w0)

            @pl.when(b == 1)
            def _():
                pltpu.async_copy(
                    rows1, out.at[pl.ds(e0, BTG), pl.ds(colbase, 48)], sem_w1)

            @pl.when((i + 1 < NI_G) & (b == 0))
            def _():
                idx_load(i + 2, 0)

            @pl.when((i + 1 < NI_G) & (b == 1))
            def _():
                idx_load(i + 2, 1)
            return carry

        lax.fori_loop(0, NI_G, body, 0)
        for lb2 in ((NI_G - 2) % 2, (NI_G - 1) % 2):
            pltpu.make_async_copy(
                rows[lb2], out.at[pl.ds(0, BTG), pl.ds(colbase, 48)],
                sem_w[lb2]).wait()
        pltpu.make_async_copy(idxarr.at[pl.ds(0, K_G)], idx0, sem_i).wait()

    run_pass(tab_r, idx_r, 0)
    run_pass(tab_c, idx_c, 48)


@functools.cache
def _make_scatter_paired():
    scratch = [pltpu.VMEM((16, 128), jnp.int32),
               pltpu.VMEM((BE // 2, 16), f32),
               pltpu.VMEM((ZCH, 16), f32),
               pltpu.VMEM((128, 16), f32),
               pltpu.VMEM_SHARED((N_PAD, 16), f32),
               pltpu.SemaphoreType.DMA]

    @functools.partial(
        pl.kernel,
        out_type=[jax.ShapeDtypeStruct((NC, 4, N_PAD, 16), f32)],
        mesh=_sc_mesh(),
        scratch_types=scratch,
        compiler_params=_SC_PARAMS,
    )
    def _scatter_kernel(idx_s, vals, out, idx_v, vals_v, zbuf, ones_v, acc,
                        sem):
        cid = lax.axis_index("c")
        sid = lax.axis_index("s")
        wid = cid * NS + sid
        tbase = wid * NB_T

        def zfill(i, carry):
            zbuf[i, :] = jnp.zeros((16,), f32)
            return carry
        lax.fori_loop(0, ZCH, zfill, 0)

        def ofill(i, carry):
            ones_v[i, :] = jnp.full((16,), 1.0, f32)
            return carry
        lax.fori_loop(0, 128, ofill, 0)

        for p in range(4):
            is_deg = p == 3

            def zero(z, carry):
                pltpu.sync_copy(zbuf, acc.at[pl.ds(sid * TROWS + z * ZCH, ZCH)])
                return carry
            lax.fori_loop(0, 16, zero, 0)
            plsc.subcore_barrier()

            def body(i, carry):
                gb = tbase + i
                pltpu.sync_copy(idx_s.at[pl.ds(16 * gb, 16)], idx_v)
                if not is_deg:
                    pltpu.sync_copy(
                        vals.at[pl.ds(BE // 2 * gb, BE // 2),
                                pl.ds(16 * p, 16)], vals_v)
                for k in range(8):
                    src = ones_v if is_deg else vals_v.at[pl.ds(k * 128, 128)]
                    pltpu.sync_copy(src, acc.at[idx_v.at[k]], add=True)
                if not is_deg:
                    pltpu.sync_copy(
                        vals.at[pl.ds(BE // 2 * gb, BE // 2),
                                pl.ds(64 + 16 * p, 16)], vals_v)
                for k in range(8):
                    src = ones_v if is_deg else vals_v.at[pl.ds(k * 128, 128)]
                    pltpu.sync_copy(src, acc.at[idx_v.at[8 + k]], add=True)
                return carry
            lax.fori_loop(0, NB_T, body, 0)
            plsc.subcore_barrier()
            pltpu.sync_copy(acc.at[pl.ds(sid * TROWS, TROWS)],
                            out.at[cid, p, pl.ds(sid * TROWS, TROWS)])
            plsc.subcore_barrier()

    return _scatter_kernel


@functools.cache
def _make_scatter(nvals, with_deg):
    npass = nvals + (1 if with_deg else 0)
    scratch = [pltpu.VMEM((KS_S, 128), jnp.int32),
               pltpu.VMEM((BTS, 16), f32),
               pltpu.VMEM((ZCH, 16), f32),
               pltpu.VMEM((128, 16), f32),
               pltpu.VMEM_SHARED((N_PAD, 16), f32),
               pltpu.SemaphoreType.DMA]

    @functools.partial(
        pl.kernel,
        out_type=[jax.ShapeDtypeStruct((NC, npass, N_PAD, 16), f32)],
        mesh=_sc_mesh(),
        scratch_types=scratch,
        compiler_params=_SC_PARAMS,
    )
    def _scatter_kernel(*args):
        idx_s = args[0]
        vals_hbm = args[1:1 + nvals]
        out = args[1 + nvals]
        idx_v, vals_v, zbuf, ones_v, acc, sem = args[2 + nvals:]
        cid = lax.axis_index("c")
        sid = lax.axis_index("s")
        wid = cid * NS + sid
        rbase = wid * (PER_TILE // 128)
        ebase = wid * PER_TILE

        def zfill(i, carry):
            zbuf[i, :] = jnp.zeros((16,), f32)
            return carry
        lax.fori_loop(0, ZCH, zfill, 0)
        if with_deg:
            def ofill(i, carry):
                ones_v[i, :] = jnp.full((16,), 1.0, f32)
                return carry
            lax.fori_loop(0, 128, ofill, 0)

        for p in range(npass):
            is_deg = with_deg and p == nvals

            def zero(z, carry):
                pltpu.sync_copy(zbuf, acc.at[pl.ds(sid * TROWS + z * ZCH, ZCH)])
                return carry
            lax.fori_loop(0, 16, zero, 0)
            plsc.subcore_barrier()

            def body(i, carry):
                r0 = rbase + i * KS_S
                e0 = ebase + i * BTS
                pltpu.sync_copy(idx_s.at[pl.ds(r0, KS_S)], idx_v)
                if not is_deg:
                    pltpu.sync_copy(vals_hbm[p].at[pl.ds(e0, BTS)], vals_v)
                for j in range(KS_S):
                    src = ones_v if is_deg else vals_v.at[pl.ds(j * 128, 128)]
                    pltpu.sync_copy(src, acc.at[idx_v.at[j]], add=True)
                return carry
            lax.fori_loop(0, NI_S, body, 0)
            plsc.subcore_barrier()
            pltpu.sync_copy(acc.at[pl.ds(sid * TROWS, TROWS)],
                            out.at[cid, p, pl.ds(sid * TROWS, TROWS)])
            plsc.subcore_barrier()

    return _scatter_kernel


def kernel(node_attrs, positions, edge_index, params):
    prm = params
    l0, l1 = prm["layers"][0], prm["layers"][1]

    ones_col = jnp.ones((N_NODES, 1), f32)
    na = jnp.pad(jnp.concatenate([node_attrs.astype(f32), ones_col], axis=1),
                 ((0, N_PAD - N_NODES), (0, 4)))
    pos = jnp.pad(jnp.concatenate([positions.astype(f32), ones_col], axis=1),
                  ((0, N_PAD - N_NODES), (0, 4)))

    row = edge_index[0].astype(jnp.int32)
    col = edge_index[1].astype(jnp.int32)
    pad_e = E_PAD - E_EDGES
    row_g = jnp.concatenate([row, jnp.zeros((pad_e,), jnp.int32)])
    col_g = jnp.concatenate([col, jnp.zeros((pad_e,), jnp.int32)])
    row_s = jnp.concatenate(
        [row, jnp.full((pad_e,), N_NODES, jnp.int32)])
    row_g = row_g.reshape(E_PAD // 128, 128)
    col_g = col_g.reshape(E_PAD // 128, 128)
    row_s = row_s.reshape(E_PAD // 128, 128)

    def vpad(v, rows=8):
        out = jnp.zeros((rows, v.shape[1]), f32)
        return out.at[:v.shape[0]].set(v)

    einT = vpad(jnp.concatenate([prm["emb_in_w"].T,
                                 prm["emb_in_b"][None, :]], axis=0))
    projT = vpad(prm["proj_w"].T)
    outWT = jnp.zeros((16, 8), f32).at[:, :3].set(prm["out_w"].T)

    def layer_mats(lp):
        w1 = lp["edge_w1"]
        w1aT = w1[:, :32].T
        w1bT = w1[:, 32:64].T
        w1dT = vpad(w1[:, 65:68].T)
        vecs = jnp.zeros((8, 32), f32)
        vecs = vecs.at[0].set(lp["edge_b1"])
        vecs = vecs.at[1].set(lp["edge_b2"])
        vecs = vecs.at[2].set(lp["coord_b1"])
        vecs = vecs.at[3].set(w1[:, 64])
        vecs = vecs.at[4].set(lp["coord_w2"][0])
        return w1aT, w1bT, w1dT, vecs, lp["edge_w2"].T, lp["coord_w1"].T

    w1aT0, w1bT0, w1dT0, vecs0, w2T0, wc1T0 = layer_mats(l0)
    w1aT1, w1bT1, w1dT1, vecs1, w2T1, wc1T1 = layer_mats(l1)
    wn1T0 = l0["node_w1"].T
    wn2T0 = l0["node_w2"].T
    nvecs0 = jnp.zeros((8, 32), f32)
    nvecs0 = nvecs0.at[0].set(l0["node_b1"]).at[1].set(l0["node_b2"])

    gr_tab0, gc_tab0, h0, x0 = _prep_call(
        na, pos, einT, projT, w1aT0, w1bT0, w1dT0)
    gather = _make_gather()
    (ge0,) = gather(gr_tab0, gc_tab0, row_g, col_g)
    (rec0,) = _edge_call(ge0, ge0, w2T0, wc1T0, vecs0)
    (p0,) = _make_scatter_paired()(row_s, rec0)
    gr_tab1, gc_tab1, x1, deg = _node_call(
        h0, x0, pos, p0, wn1T0, wn2T0, nvecs0, w1aT1, w1bT1, w1dT1)
    (ge1,) = gather(gr_tab1, gc_tab1, row_g, col_g)
    (tr1,) = _edge_last_call(ge1, ge1, w2T1, wc1T1, vecs1)
    (p1,) = _make_scatter(1, False)(row_s, tr1)
    (out,) = _final_call(x1, p1, deg, outWT)
    return out[:N_NODES, :3]
